# baseline (speedup 1.0000x reference)
import jax
import jax.numpy as jnp
from jax.experimental import pallas as pl
from jax.experimental.pallas import tpu as pltpu

N = 5000
D_IN = 12544
D_HID = 1024
BK = 256
NK = D_IN // BK
EPI_BM = 1000


def _body(x_ref, w1_ref, b1_ref, w2_ref, b2_ref, wh_ref, bh_ref,
          out_ref, acc_ref):
    k = pl.program_id(0)
    part = jnp.dot(x_ref[...], w1_ref[...],
                   preferred_element_type=jnp.float32)

    @pl.when(k == 0)
    def _():
        acc_ref[...] = part

    @pl.when(k > 0)
    def _():
        acc_ref[...] += part

    @pl.when(k == NK - 1)
    def _():
        for i in range(N // EPI_BM):
            rows = pl.ds(i * EPI_BM, EPI_BM)
            h1 = jnp.maximum(acc_ref[rows, :] + b1_ref[...], 0.0)
            h2 = jnp.maximum(
                jnp.dot(h1.astype(jnp.bfloat16), w2_ref[...],
                        preferred_element_type=jnp.float32)
                + b2_ref[...], 0.0)
            out_ref[rows, :] = (
                jnp.dot(h2.astype(jnp.bfloat16), wh_ref[...],
                        preferred_element_type=jnp.float32)
                + bh_ref[...])


def kernel(feature_vectors, W1, b1, W2, b2, Wc, bc, Wr, br):
    wh = jnp.concatenate([Wc, Wr], axis=1).astype(jnp.bfloat16)
    bh = jnp.concatenate([bc, br])[None, :]
    w2 = W2.astype(jnp.bfloat16)
    b1r = b1[None, :]
    b2r = b2[None, :]
    n_heads = wh.shape[1]

    out = pl.pallas_call(
        _body,
        grid=(NK,),
        in_specs=[
            pl.BlockSpec((N, BK), lambda k: (0, k)),
            pl.BlockSpec((BK, D_HID), lambda k: (k, 0)),
            pl.BlockSpec((1, D_HID), lambda k: (0, 0)),
            pl.BlockSpec((D_HID, D_HID), lambda k: (0, 0)),
            pl.BlockSpec((1, D_HID), lambda k: (0, 0)),
            pl.BlockSpec((D_HID, n_heads), lambda k: (0, 0)),
            pl.BlockSpec((1, n_heads), lambda k: (0, 0)),
        ],
        out_specs=pl.BlockSpec((N, n_heads), lambda k: (0, 0)),
        out_shape=jax.ShapeDtypeStruct((N, n_heads), jnp.float32),
        scratch_shapes=[pltpu.VMEM((N, D_HID), jnp.float32)],
        compiler_params=pltpu.CompilerParams(
            vmem_limit_bytes=100 * 1024 * 1024),
    )(feature_vectors, W1, b1r, w2, b2r, wh, bh)

    return out[:, :4], out[:, 4:]


# speedup vs baseline: 1.4003x; 1.4003x over previous
import jax
import jax.numpy as jnp
from jax.experimental import pallas as pl
from jax.experimental.pallas import tpu as pltpu

N = 5000
D_IN = 12544
D_HID = 1024
BM = 1000
BK = 1792
NM = N // BM
NK = D_IN // BK
EPI_BM = 1000


def _body(x_ref, w1_ref, b1_ref, w2_ref, b2_ref, wh_ref, bh_ref,
          out_ref, acc_ref):
    k = pl.program_id(0)
    m = pl.program_id(1)
    part = jnp.dot(x_ref[...], w1_ref[...],
                   preferred_element_type=jnp.float32)

    @pl.when(k == 0)
    def _():
        acc_ref[pl.ds(m * BM, BM), :] = part.astype(jnp.bfloat16)

    @pl.when(k > 0)
    def _():
        rows = pl.ds(m * BM, BM)
        acc_ref[rows, :] = (acc_ref[rows, :].astype(jnp.float32)
                            + part).astype(jnp.bfloat16)

    @pl.when(k == NK - 1)
    def _():
        for i in range(BM // EPI_BM):
            rows = pl.ds(m * BM + i * EPI_BM, EPI_BM)
            orows = pl.ds(i * EPI_BM, EPI_BM)
            h1 = jnp.maximum(
                acc_ref[rows, :].astype(jnp.float32) + b1_ref[...], 0.0)
            h2 = jnp.maximum(
                jnp.dot(h1.astype(jnp.bfloat16), w2_ref[...],
                        preferred_element_type=jnp.float32)
                + b2_ref[...], 0.0)
            out_ref[orows, :] = (
                jnp.dot(h2.astype(jnp.bfloat16), wh_ref[...],
                        preferred_element_type=jnp.float32)
                + bh_ref[...])


def kernel(feature_vectors, W1, b1, W2, b2, Wc, bc, Wr, br):
    wh = jnp.concatenate([Wc, Wr], axis=1).astype(jnp.bfloat16)
    bh = jnp.concatenate([bc, br])[None, :]
    w2 = W2.astype(jnp.bfloat16)
    b1r = b1[None, :]
    b2r = b2[None, :]
    n_heads = wh.shape[1]

    out = pl.pallas_call(
        _body,
        grid=(NK, NM),
        in_specs=[
            pl.BlockSpec((BM, BK), lambda k, m: (m, k)),
            pl.BlockSpec((BK, D_HID), lambda k, m: (k, 0)),
            pl.BlockSpec((1, D_HID), lambda k, m: (0, 0)),
            pl.BlockSpec((D_HID, D_HID), lambda k, m: (0, 0)),
            pl.BlockSpec((1, D_HID), lambda k, m: (0, 0)),
            pl.BlockSpec((D_HID, n_heads), lambda k, m: (0, 0)),
            pl.BlockSpec((1, n_heads), lambda k, m: (0, 0)),
        ],
        out_specs=pl.BlockSpec((BM, n_heads), lambda k, m: (m, 0)),
        out_shape=jax.ShapeDtypeStruct((N, n_heads), jnp.float32),
        scratch_shapes=[pltpu.VMEM((N, D_HID), jnp.bfloat16)],
        compiler_params=pltpu.CompilerParams(
            vmem_limit_bytes=100 * 1024 * 1024),
    )(feature_vectors, W1, b1r, w2, b2r, wh, bh)

    return out[:, :4], out[:, 4:]
